# baseline (speedup 1.0000x reference)
import functools

import jax
import jax.numpy as jnp
from jax import lax
from jax.experimental import pallas as pl
from jax.experimental.pallas import tpu as pltpu
from jax.experimental.pallas import tpu_sc as plsc

NUM_CORES = 2
NUM_SUBCORES = 16
NW = NUM_CORES * NUM_SUBCORES

SEQ_ROWS = 16384
SEQ_COLS = 50
D = 64
V = 1000000
B = SEQ_ROWS * SEQ_COLS
B_PER_W = B // NW
CHUNK = 256
N_CHUNKS = B_PER_W // CHUNK
DP = 128
VBLK = 2048


def _wtr_body(in_ref, out_ref):
    out_ref[:, :D] = in_ref[...].T


def _body(seq_hbm, table_hbm, out_hbm, idx_v, rows_v, gsem0, gsem1, ssem0,
          ssem1):
    wid = lax.axis_index("s") * NUM_CORES + lax.axis_index("c")
    base = wid * B_PER_W

    pltpu.sync_copy(seq_hbm.at[pl.ds(base, B_PER_W)], idx_v)

    def g_copy(i, b, sem):
        return pltpu.make_async_copy(
            table_hbm.at[idx_v.at[pl.ds(i * CHUNK, CHUNK)]],
            rows_v.at[b], sem)

    def s_copy(i, b, sem):
        return pltpu.make_async_copy(
            rows_v.at[b, :, pl.ds(0, D)],
            out_hbm.at[pl.ds(base + i * CHUNK, CHUNK)], sem)

    N, P = N_CHUNKS, N_CHUNKS // 2

    g_copy(0, 0, gsem0).start()
    g_copy(0, 0, gsem0).wait()
    s_copy(0, 0, ssem0).start()
    g_copy(1, 1, gsem1).start()
    g_copy(1, 1, gsem1).wait()
    s_copy(1, 1, ssem1).start()
    s_copy(0, 0, ssem0).wait()
    g_copy(2, 0, gsem0).start()

    def pair(p, carry):
        i0 = 2 * p
        g_copy(i0, 0, gsem0).wait()
        s_copy(i0, 0, ssem0).start()
        s_copy(i0 - 1, 1, ssem1).wait()
        g_copy(i0 + 1, 1, gsem1).start()
        g_copy(i0 + 1, 1, gsem1).wait()
        s_copy(i0 + 1, 1, ssem1).start()
        s_copy(i0, 0, ssem0).wait()
        g_copy(i0 + 2, 0, gsem0).start()
        return carry

    lax.fori_loop(1, P - 1, pair, 0)

    i0 = N - 2
    g_copy(i0, 0, gsem0).wait()
    s_copy(i0, 0, ssem0).start()
    s_copy(i0 - 1, 1, ssem1).wait()
    g_copy(i0 + 1, 1, gsem1).start()
    g_copy(i0 + 1, 1, gsem1).wait()
    s_copy(i0 + 1, 1, ssem1).start()
    s_copy(i0, 0, ssem0).wait()
    s_copy(i0 + 1, 1, ssem1).wait()


@jax.jit
def _embed(sequence, weight):
    mesh = plsc.VectorSubcoreMesh(
        core_axis_name="c", subcore_axis_name="s",
        num_cores=NUM_CORES, num_subcores=NUM_SUBCORES)

    gather_k = pl.kernel(
        _body,
        out_type=jax.ShapeDtypeStruct((B, D), jnp.float32),
        mesh=mesh,
        scratch_types=[
            pltpu.VMEM((B_PER_W,), jnp.int32),
            pltpu.VMEM((2, CHUNK, DP), jnp.float32),
            pltpu.SemaphoreType.DMA,
            pltpu.SemaphoreType.DMA,
            pltpu.SemaphoreType.DMA,
            pltpu.SemaphoreType.DMA,
        ],
        compiler_params=pltpu.CompilerParams(use_tc_tiling_on_sc=False),
    )
    seq_t = sequence.T.reshape(-1)
    wpad = pl.pallas_call(
        _wtr_body,
        grid=(pl.cdiv(V, VBLK),),
        in_specs=[pl.BlockSpec((D, VBLK), lambda i: (0, i))],
        out_specs=pl.BlockSpec((VBLK, DP), lambda i: (i, 0)),
        out_shape=jax.ShapeDtypeStruct((V, DP), jnp.float32),
    )(weight.T)
    out = gather_k(seq_t, wpad)
    return jnp.swapaxes(out.reshape(SEQ_COLS, SEQ_ROWS, D), 0, 1)


def kernel(sequence, weight):
    return _embed(sequence, weight)


# speedup vs baseline: 1.0112x; 1.0112x over previous
import jax
import jax.numpy as jnp
from jax import lax
from jax.experimental import pallas as pl
from jax.experimental.pallas import tpu as pltpu
from jax.experimental.pallas import tpu_sc as plsc

NUM_CORES = 2
NUM_SUBCORES = 16
NW = NUM_CORES * NUM_SUBCORES

SEQ_ROWS = 16384
SEQ_COLS = 50
D = 64
V = 1000000
B = SEQ_ROWS * SEQ_COLS
B_PER_W = B // NW
CHUNK = 512
N_CHUNKS = B_PER_W // CHUNK


def _body(seq_hbm, table_hbm, out_hbm, idx_v, rows_v, gsem0, gsem1, ssem0,
          ssem1):
    wid = lax.axis_index("s") * NUM_CORES + lax.axis_index("c")
    base = wid * B_PER_W

    pltpu.sync_copy(seq_hbm.at[pl.ds(base, B_PER_W)], idx_v)

    def g_copy(i, b, sem):
        return pltpu.make_async_copy(
            table_hbm.at[idx_v.at[pl.ds(i * CHUNK, CHUNK)]],
            rows_v.at[b], sem)

    def s_copy(i, b, sem):
        return pltpu.make_async_copy(
            rows_v.at[b], out_hbm.at[pl.ds(base + i * CHUNK, CHUNK)], sem)

    N, P = N_CHUNKS, N_CHUNKS // 2

    g_copy(0, 0, gsem0).start()
    g_copy(0, 0, gsem0).wait()
    s_copy(0, 0, ssem0).start()
    g_copy(1, 1, gsem1).start()
    g_copy(1, 1, gsem1).wait()
    s_copy(1, 1, ssem1).start()
    s_copy(0, 0, ssem0).wait()
    g_copy(2, 0, gsem0).start()

    def pair(p, carry):
        i0 = 2 * p
        g_copy(i0, 0, gsem0).wait()
        s_copy(i0, 0, ssem0).start()
        s_copy(i0 - 1, 1, ssem1).wait()
        g_copy(i0 + 1, 1, gsem1).start()
        g_copy(i0 + 1, 1, gsem1).wait()
        s_copy(i0 + 1, 1, ssem1).start()
        s_copy(i0, 0, ssem0).wait()
        g_copy(i0 + 2, 0, gsem0).start()
        return carry

    lax.fori_loop(1, P - 1, pair, 0)

    i0 = N - 2
    g_copy(i0, 0, gsem0).wait()
    s_copy(i0, 0, ssem0).start()
    s_copy(i0 - 1, 1, ssem1).wait()
    g_copy(i0 + 1, 1, gsem1).start()
    g_copy(i0 + 1, 1, gsem1).wait()
    s_copy(i0 + 1, 1, ssem1).start()
    s_copy(i0, 0, ssem0).wait()
    s_copy(i0 + 1, 1, ssem1).wait()


@jax.jit
def _embed(sequence, weight):
    mesh = plsc.VectorSubcoreMesh(
        core_axis_name="c", subcore_axis_name="s",
        num_cores=NUM_CORES, num_subcores=NUM_SUBCORES)

    gather_k = pl.kernel(
        _body,
        out_type=jax.ShapeDtypeStruct((B, D), jnp.float32),
        mesh=mesh,
        scratch_types=[
            pltpu.VMEM((B_PER_W,), jnp.int32),
            pltpu.VMEM((2, CHUNK, D), jnp.float32),
            pltpu.SemaphoreType.DMA,
            pltpu.SemaphoreType.DMA,
            pltpu.SemaphoreType.DMA,
            pltpu.SemaphoreType.DMA,
        ],
        compiler_params=pltpu.CompilerParams(use_tc_tiling_on_sc=False),
    )
    out = gather_k(sequence.reshape(-1), weight)
    return out.reshape(SEQ_ROWS, SEQ_COLS, D)


def kernel(sequence, weight):
    return _embed(sequence, weight)


# speedup vs baseline: 1.0520x; 1.0403x over previous
import jax
import jax.numpy as jnp
from jax import lax
from jax.experimental import pallas as pl
from jax.experimental.pallas import tpu as pltpu
from jax.experimental.pallas import tpu_sc as plsc

NUM_CORES = 2
NUM_SUBCORES = 16
NW = NUM_CORES * NUM_SUBCORES

SEQ_ROWS = 16384
SEQ_COLS = 50
D = 64
V = 1000000
B = SEQ_ROWS * SEQ_COLS
B_PER_W = B // NW
CHUNK = 512
N_CHUNKS = B_PER_W // CHUNK
VBLK = 2048


def _wtr_body(in_ref, out_ref):
    x = in_ref[...].T.reshape(VBLK // 2, 2, D)
    out_ref[...] = jnp.concatenate([x[:, 0, :], x[:, 1, :]], axis=1)


def _body(seq_hbm, table_hbm, out_hbm, idx_v, rows_v, gsem0, gsem1, ssem0,
          ssem1):
    wid = lax.axis_index("s") * NUM_CORES + lax.axis_index("c")
    base = wid * B_PER_W

    pltpu.sync_copy(seq_hbm.at[pl.ds(base, B_PER_W)], idx_v)

    def g_copy(i, b, sem):
        return pltpu.make_async_copy(
            table_hbm.at[idx_v.at[pl.ds(i * CHUNK, CHUNK)]],
            rows_v.at[b], sem)

    def s_copy(i, b, sem):
        return pltpu.make_async_copy(
            rows_v.at[b], out_hbm.at[pl.ds(base + i * CHUNK, CHUNK)], sem)

    N, P = N_CHUNKS, N_CHUNKS // 2

    g_copy(0, 0, gsem0).start()
    g_copy(0, 0, gsem0).wait()
    s_copy(0, 0, ssem0).start()
    g_copy(1, 1, gsem1).start()
    g_copy(1, 1, gsem1).wait()
    s_copy(1, 1, ssem1).start()
    s_copy(0, 0, ssem0).wait()
    g_copy(2, 0, gsem0).start()

    def pair(p, carry):
        i0 = 2 * p
        g_copy(i0, 0, gsem0).wait()
        s_copy(i0, 0, ssem0).start()
        s_copy(i0 - 1, 1, ssem1).wait()
        g_copy(i0 + 1, 1, gsem1).start()
        g_copy(i0 + 1, 1, gsem1).wait()
        s_copy(i0 + 1, 1, ssem1).start()
        s_copy(i0, 0, ssem0).wait()
        g_copy(i0 + 2, 0, gsem0).start()
        return carry

    lax.fori_loop(1, P - 1, pair, 0)

    i0 = N - 2
    g_copy(i0, 0, gsem0).wait()
    s_copy(i0, 0, ssem0).start()
    s_copy(i0 - 1, 1, ssem1).wait()
    g_copy(i0 + 1, 1, gsem1).start()
    g_copy(i0 + 1, 1, gsem1).wait()
    s_copy(i0 + 1, 1, ssem1).start()
    s_copy(i0, 0, ssem0).wait()
    s_copy(i0 + 1, 1, ssem1).wait()


@jax.jit
def _embed(sequence, weight):
    mesh = plsc.VectorSubcoreMesh(
        core_axis_name="c", subcore_axis_name="s",
        num_cores=NUM_CORES, num_subcores=NUM_SUBCORES)

    gather_k = pl.kernel(
        _body,
        out_type=jax.ShapeDtypeStruct((B, D), jnp.float32),
        mesh=mesh,
        scratch_types=[
            pltpu.VMEM((B_PER_W,), jnp.int32),
            pltpu.VMEM((2, CHUNK, D), jnp.float32),
            pltpu.SemaphoreType.DMA,
            pltpu.SemaphoreType.DMA,
            pltpu.SemaphoreType.DMA,
            pltpu.SemaphoreType.DMA,
        ],
        compiler_params=pltpu.CompilerParams(use_tc_tiling_on_sc=False),
    )
    wpacked = pl.pallas_call(
        _wtr_body,
        grid=(pl.cdiv(V, VBLK),),
        in_specs=[pl.BlockSpec((D, VBLK), lambda i: (0, i))],
        out_specs=pl.BlockSpec((VBLK // 2, 2 * D), lambda i: (i, 0)),
        out_shape=jax.ShapeDtypeStruct((V // 2, 2 * D), jnp.float32),
    )(weight.T)
    out = gather_k(sequence.reshape(-1), wpacked.reshape(V, D))
    return out.reshape(SEQ_ROWS, SEQ_COLS, D)


def kernel(sequence, weight):
    return _embed(sequence, weight)
